# baseline (speedup 1.0000x reference)
import functools

import jax
import jax.numpy as jnp
from jax import lax
from jax.experimental import pallas as pl
from jax.experimental.pallas import tpu as pltpu
from jax.experimental.pallas import tpu_sc as plsc


@functools.lru_cache(maxsize=None)
def _gather_kernel(B, V, D, NC, NS):
    NW = NC * NS
    b_per_w = B // NW
    d0 = D // 8
    NSLOT = 8
    mesh = plsc.VectorSubcoreMesh(core_axis_name="c", subcore_axis_name="s")

    @functools.partial(
        pl.kernel,
        mesh=mesh,
        compiler_params=pltpu.CompilerParams(needs_layout_passes=False),
        out_type=jax.ShapeDtypeStruct((2, D, B), jnp.float32),
        scratch_types=[
            pltpu.VMEM((b_per_w,), jnp.int32),
            pltpu.VMEM((b_per_w,), jnp.int32),
            pltpu.VMEM((NSLOT, D, 128), jnp.float32),
            pltpu.VMEM((NSLOT, D, 128), jnp.float32),
            pltpu.VMEM((D, b_per_w), jnp.float32),
            pltpu.VMEM((D, b_per_w), jnp.float32),
            pltpu.SemaphoreType.DMA,
            pltpu.SemaphoreType.DMA,
        ],
    )
    def k(in_idx, out_idx, in_t, out_t, out, sidx0, sidx1, tb0, tb1,
          st0, st1, sem0, sem1):
        wid = lax.axis_index("s") * NC + lax.axis_index("c")
        base = wid * b_per_w

        pltpu.sync_copy(in_idx.at[pl.ds(base, b_per_w)], sidx0)
        pltpu.sync_copy(out_idx.at[pl.ds(base, b_per_w)], sidx1)

        lo16 = lax.iota(jnp.int32, 16)
        hi16 = lo16 + 16

        def fire(tab, tb, sem, slot, v):
            col = pl.multiple_of((v >> 7) * 128, 128)
            for p in range(d0):
                pltpu.async_copy(
                    tab.at[p, :, pl.ds(col, 128)],
                    tb.at[slot, pl.ds(p * 8, 8), :], sem)

        def drain(tab, tb, sem, slot):
            for p in range(d0):
                pltpu.make_async_copy(
                    tab.at[p, :, pl.ds(0, 128)],
                    tb.at[slot, pl.ds(p * 8, 8), :], sem).wait()

        def make_chunk(sidx, tab, tb, st, sem):
            def chunk(c, carry):
                vec = sidx[pl.ds(c * 16, 16)]
                for h in range(2):
                    for j in range(NSLOT):
                        fire(tab, tb, sem, j, vec[h * NSLOT + j])
                    for j in range(NSLOT):
                        drain(tab, tb, sem, j)
                    for j in range(NSLOT):
                        i = c * 16 + h * NSLOT + j
                        lane = vec[h * NSLOT + j] & 127
                        lane_v = jnp.broadcast_to(lane, (16,))
                        i_v = jnp.broadcast_to(i, (16,))
                        row = tb.at[j]
                        vals_lo = plsc.load_gather(row, [lo16, lane_v])
                        vals_hi = plsc.load_gather(row, [hi16, lane_v])
                        plsc.store_scatter(st, [lo16, i_v], vals_lo)
                        plsc.store_scatter(st, [hi16, i_v], vals_hi)
                return carry
            return chunk

        n_chunks = b_per_w // 16
        lax.fori_loop(0, n_chunks, make_chunk(sidx0, in_t, tb0, st0, sem0), 0)
        lax.fori_loop(0, n_chunks, make_chunk(sidx1, out_t, tb1, st1, sem1), 0)

        pltpu.sync_copy(st0, out.at[0, :, pl.ds(base, b_per_w)])
        pltpu.sync_copy(st1, out.at[1, :, pl.ds(base, b_per_w)])

    return k


def kernel(input_words, output_words, in_table, out_table):
    B = input_words.shape[0]
    V, D = in_table.shape
    info = plsc.get_sparse_core_info()
    k = _gather_kernel(B, V, D, info.num_cores, info.num_subcores)
    res = k(input_words.astype(jnp.int32), output_words.astype(jnp.int32),
            in_table.T.reshape(D // 8, 8, V), out_table.T.reshape(D // 8, 8, V))
    return res.transpose(0, 2, 1)


# speedup vs baseline: 1.1591x; 1.1591x over previous
import functools

import jax
import jax.numpy as jnp
from jax import lax
from jax.experimental import pallas as pl
from jax.experimental.pallas import tpu as pltpu
from jax.experimental.pallas import tpu_sc as plsc


@functools.lru_cache(maxsize=None)
def _gather_kernel(B, V, D, NC, NS):
    NW = NC * NS
    b_per_w = B // NW
    d0 = D // 8
    NSLOT = 8
    mesh = plsc.VectorSubcoreMesh(core_axis_name="c", subcore_axis_name="s")

    @functools.partial(
        pl.kernel,
        mesh=mesh,
        compiler_params=pltpu.CompilerParams(needs_layout_passes=False),
        out_type=jax.ShapeDtypeStruct((2, D, B), jnp.float32),
        scratch_types=[
            pltpu.VMEM((b_per_w,), jnp.int32),
            pltpu.VMEM((b_per_w,), jnp.int32),
            pltpu.VMEM((NSLOT, D, 128), jnp.float32),
            pltpu.VMEM((NSLOT, D, 128), jnp.float32),
            pltpu.VMEM((D, b_per_w), jnp.float32),
            pltpu.VMEM((D, b_per_w), jnp.float32),
            pltpu.SemaphoreType.DMA,
            pltpu.SemaphoreType.DMA,
        ],
    )
    def k(in_idx, out_idx, in_t, out_t, out, sidx0, sidx1, tb0, tb1,
          st0, st1, sem0, sem1):
        wid = lax.axis_index("s") * NC + lax.axis_index("c")
        base = wid * b_per_w

        pltpu.sync_copy(in_idx.at[pl.ds(base, b_per_w)], sidx0)
        pltpu.sync_copy(out_idx.at[pl.ds(base, b_per_w)], sidx1)

        lo16 = lax.iota(jnp.int32, 16)
        hi16 = lo16 + 16

        def fire(tab, tb, sem, slot, v):
            col = pl.multiple_of((v >> 7) * 128, 128)
            for p in range(d0):
                pltpu.async_copy(
                    tab.at[p, :, pl.ds(col, 128)],
                    tb.at[slot, pl.ds(p * 8, 8), :], sem)

        def drain(tab, tb, sem, slot):
            for p in range(d0):
                pltpu.make_async_copy(
                    tab.at[p, :, pl.ds(0, 128)],
                    tb.at[slot, pl.ds(p * 8, 8), :], sem).wait()

        def extract(tb, st, vec, c, h, j):
            i = c * 16 + h * NSLOT + j
            lane = vec[h * NSLOT + j] & 127
            lane_v = jnp.broadcast_to(lane, (16,))
            i_v = jnp.broadcast_to(i, (16,))
            row = tb.at[j]
            vals_lo = plsc.load_gather(row, [lo16, lane_v])
            vals_hi = plsc.load_gather(row, [hi16, lane_v])
            plsc.store_scatter(st, [lo16, i_v], vals_lo)
            plsc.store_scatter(st, [hi16, i_v], vals_hi)

        def chunk(c, carry):
            vec0 = sidx0[pl.ds(c * 16, 16)]
            vec1 = sidx1[pl.ds(c * 16, 16)]
            for h in range(2):
                for j in range(NSLOT):
                    fire(in_t, tb0, sem0, j, vec0[h * NSLOT + j])
                for j in range(NSLOT):
                    fire(out_t, tb1, sem1, j, vec1[h * NSLOT + j])
                for j in range(NSLOT):
                    drain(in_t, tb0, sem0, j)
                for j in range(NSLOT):
                    extract(tb0, st0, vec0, c, h, j)
                for j in range(NSLOT):
                    drain(out_t, tb1, sem1, j)
                for j in range(NSLOT):
                    extract(tb1, st1, vec1, c, h, j)
            return carry

        n_chunks = b_per_w // 16
        lax.fori_loop(0, n_chunks, chunk, 0)

        pltpu.sync_copy(st0, out.at[0, :, pl.ds(base, b_per_w)])
        pltpu.sync_copy(st1, out.at[1, :, pl.ds(base, b_per_w)])

    return k


def kernel(input_words, output_words, in_table, out_table):
    B = input_words.shape[0]
    V, D = in_table.shape
    info = plsc.get_sparse_core_info()
    k = _gather_kernel(B, V, D, info.num_cores, info.num_subcores)
    res = k(input_words.astype(jnp.int32), output_words.astype(jnp.int32),
            in_table.T.reshape(D // 8, 8, V), out_table.T.reshape(D // 8, 8, V))
    return res.transpose(0, 2, 1)
